# baseline (speedup 1.0000x reference)
import functools

import jax
import jax.numpy as jnp
from jax import lax
from jax.experimental import pallas as pl
from jax.experimental.pallas import tpu as pltpu

_EPS = 1e-6


def _split_hi_lo(x):
    hi = x.astype(jnp.bfloat16)
    lo = (x - hi.astype(jnp.float32)).astype(jnp.bfloat16)
    return hi, lo


def _pass1_body(inc_ref, y_ref, dvis_ref, ehi_ref, elo_ref,
                eacc_ref, deacc_ref, *, nsteps):
    i = pl.program_id(0)
    blk = inc_ref[...]
    dv = jnp.sum(blk, axis=1)
    dvis = lax.rsqrt(dv + _EPS)
    dvis_ref[0, 0, :] = dvis
    x1 = y_ref[...] * dvis[:, None]
    b16 = blk.astype(jnp.bfloat16)
    x1h, x1l = _split_hi_lo(x1)
    dn = (((0,), (0,)), ((), ()))
    part = lax.dot_general(b16, x1h, dn, preferred_element_type=jnp.float32)
    part = part + lax.dot_general(b16, x1l, dn,
                                  preferred_element_type=jnp.float32)

    @pl.when(i == 0)
    def _init():
        eacc_ref[...] = part
        deacc_ref[...] = jnp.sum(blk, axis=0, keepdims=True)

    @pl.when(i > 0)
    def _acc():
        eacc_ref[...] += part
        deacc_ref[...] += jnp.sum(blk, axis=0, keepdims=True)

    @pl.when(i == nsteps - 1)
    def _fin():
        de_inv = 1.0 / (deacc_ref[0, :] + _EPS)
        e = eacc_ref[...] * de_inv[:, None]
        ehi = e.astype(jnp.bfloat16)
        ehi_ref[...] = ehi
        elo_ref[...] = (e - ehi.astype(jnp.float32)).astype(jnp.bfloat16)


def _pass2_body(inc_ref, dvis_ref, ehi_ref, elo_ref, w_ref, b_ref, out_ref):
    blk = inc_ref[...].astype(jnp.bfloat16)
    dn = (((1,), (0,)), ((), ()))
    x2 = lax.dot_general(blk, ehi_ref[...], dn,
                         preferred_element_type=jnp.float32)
    x2 = x2 + lax.dot_general(blk, elo_ref[...], dn,
                              preferred_element_type=jnp.float32)
    x2 = x2 * dvis_ref[0, 0, :][:, None]
    x2h, x2l = _split_hi_lo(x2)
    wh, wl = _split_hi_lo(w_ref[...])
    dnm = (((1,), (0,)), ((), ()))
    z = lax.dot_general(x2h, wh, dnm, preferred_element_type=jnp.float32)
    z = z + lax.dot_general(x2l, wh, dnm, preferred_element_type=jnp.float32)
    z = z + lax.dot_general(x2h, wl, dnm, preferred_element_type=jnp.float32)
    out_ref[...] = jnp.tanh(z + b_ref[0, :][None, :])


@functools.partial(jax.jit, static_argnames=())
def kernel(t, y, incidence, W, b):
    del t
    N, M = incidence.shape
    D = y.shape[1]
    BN = 1000 if N % 1000 == 0 else N
    G = N // BN

    dvis, ehi, elo = pl.pallas_call(
        functools.partial(_pass1_body, nsteps=G),
        grid=(G,),
        in_specs=[
            pl.BlockSpec((BN, M), lambda i: (i, 0)),
            pl.BlockSpec((BN, D), lambda i: (i, 0)),
        ],
        out_specs=[
            pl.BlockSpec((1, 1, BN), lambda i: (i, 0, 0)),
            pl.BlockSpec((M, D), lambda i: (0, 0)),
            pl.BlockSpec((M, D), lambda i: (0, 0)),
        ],
        out_shape=[
            jax.ShapeDtypeStruct((G, 1, BN), jnp.float32),
            jax.ShapeDtypeStruct((M, D), jnp.bfloat16),
            jax.ShapeDtypeStruct((M, D), jnp.bfloat16),
        ],
        scratch_shapes=[
            pltpu.VMEM((M, D), jnp.float32),
            pltpu.VMEM((1, M), jnp.float32),
        ],
    )(incidence, y)

    out = pl.pallas_call(
        _pass2_body,
        grid=(G,),
        in_specs=[
            pl.BlockSpec((BN, M), lambda i: (i, 0)),
            pl.BlockSpec((1, 1, BN), lambda i: (i, 0, 0)),
            pl.BlockSpec((M, D), lambda i: (0, 0)),
            pl.BlockSpec((M, D), lambda i: (0, 0)),
            pl.BlockSpec((D, D), lambda i: (0, 0)),
            pl.BlockSpec((1, D), lambda i: (0, 0)),
        ],
        out_specs=pl.BlockSpec((BN, D), lambda i: (i, 0)),
        out_shape=jax.ShapeDtypeStruct((N, D), jnp.float32),
    )(incidence, dvis, ehi, elo, W, b.reshape(1, D))
    return out


# speedup vs baseline: 1.0274x; 1.0274x over previous
import functools

import jax
import jax.numpy as jnp
from jax import lax
from jax.experimental import pallas as pl
from jax.experimental.pallas import tpu as pltpu

_EPS = 1e-6


def _split_hi_lo(x):
    hi = x.astype(jnp.bfloat16)
    lo = (x - hi.astype(jnp.float32)).astype(jnp.bfloat16)
    return hi, lo


def _body(inc_ref, y_ref, w_ref, b_ref, out_ref,
          eacct_ref, deacc_ref, ehi_ref, elo_ref, dvis_ref, *, nsteps):
    p = pl.program_id(0)
    i = pl.program_id(1)

    @pl.when(p == 0)
    def _phase0():
        blk = inc_ref[...]
        b16 = blk.astype(jnp.bfloat16)
        dv = jnp.sum(blk, axis=1)
        dvis = lax.rsqrt(dv + _EPS)
        dvis_ref[i] = dvis[:, None]
        x1 = y_ref[...] * dvis[:, None]
        x1h, x1l = _split_hi_lo(x1)
        dn = (((1,), (0,)), ((), ()))
        part = lax.dot_general(x1h.T, b16, dn,
                               preferred_element_type=jnp.float32)
        part = part + lax.dot_general(x1l.T, b16, dn,
                                      preferred_element_type=jnp.float32)

        @pl.when(i == 0)
        def _init():
            eacct_ref[...] = part
            deacc_ref[...] = jnp.sum(blk, axis=0, keepdims=True)

        @pl.when(i > 0)
        def _acc():
            eacct_ref[...] += part
            deacc_ref[...] += jnp.sum(blk, axis=0, keepdims=True)

        @pl.when(i == nsteps - 1)
        def _fin():
            de_inv = 1.0 / (deacc_ref[...] + _EPS)
            et = eacct_ref[...] * de_inv
            eth, etl = _split_hi_lo(et)
            ehi_ref[...] = eth.T
            elo_ref[...] = etl.T

    @pl.when(p == 1)
    def _phase1():
        blk = inc_ref[...].astype(jnp.bfloat16)
        dn = (((1,), (0,)), ((), ()))
        x2 = lax.dot_general(blk, ehi_ref[...], dn,
                             preferred_element_type=jnp.float32)
        x2 = x2 + lax.dot_general(blk, elo_ref[...], dn,
                                  preferred_element_type=jnp.float32)
        x2 = x2 * dvis_ref[i]
        x2h, x2l = _split_hi_lo(x2)
        wh, wl = _split_hi_lo(w_ref[...])
        z = lax.dot_general(x2h, wh, dn, preferred_element_type=jnp.float32)
        z = z + lax.dot_general(x2l, wh, dn, preferred_element_type=jnp.float32)
        z = z + lax.dot_general(x2h, wl, dn, preferred_element_type=jnp.float32)
        out_ref[...] = jnp.tanh(z + b_ref[...])


@jax.jit
def kernel(t, y, incidence, W, b):
    del t
    N, M = incidence.shape
    D = y.shape[1]
    BN = 1000 if N % 1000 == 0 else N
    G = N // BN

    out = pl.pallas_call(
        functools.partial(_body, nsteps=G),
        grid=(2, G),
        in_specs=[
            pl.BlockSpec((BN, M), lambda p, i: (i, 0)),
            pl.BlockSpec((BN, D), lambda p, i: (i, 0)),
            pl.BlockSpec((D, D), lambda p, i: (0, 0)),
            pl.BlockSpec((1, D), lambda p, i: (0, 0)),
        ],
        out_specs=pl.BlockSpec((BN, D),
                               lambda p, i: (jnp.where(p == 0, 0, i), 0)),
        out_shape=jax.ShapeDtypeStruct((N, D), jnp.float32),
        scratch_shapes=[
            pltpu.VMEM((D, M), jnp.float32),
            pltpu.VMEM((1, M), jnp.float32),
            pltpu.VMEM((M, D), jnp.bfloat16),
            pltpu.VMEM((M, D), jnp.bfloat16),
            pltpu.VMEM((G, BN, 1), jnp.float32),
        ],
    )(incidence, y, W, b.reshape(1, D))
    return out
